# Initial kernel scaffold; baseline (speedup 1.0000x reference)
import jax
import jax.numpy as jnp
from jax.experimental import pallas as pl


def kernel(x, W1, b1, g1, be1, W2, b2, g2, be2, Wm, bm, edge_index, node2graph):
    raise NotImplementedError("write your pallas kernel here")


# speedup vs baseline: 3.5293x; 3.5293x over previous
import functools

import jax
import jax.numpy as jnp
from jax import lax
from jax.experimental import pallas as pl
from jax.experimental.pallas import tpu as pltpu
from jax.experimental.pallas import tpu_sc as plsc

N = 10000
NP = 10240
E = 160000
D = 256
DH = 128
G = 64
NB = 1000
GRID = N // NB

NSC = 16
EPT = E // NSC
CHUNK = 128
CPT = -(-EPT // CHUNK)
EPTP = CPT * CHUNK
RPT = NP // NSC

_MESH = dict(core_axis_name="c", subcore_axis_name="s", num_cores=2,
             num_subcores=NSC)


def _pad_idx(a, fill):
  a = a.reshape(NSC, EPT)
  pad = jnp.full((NSC, EPTP - EPT), fill, jnp.int32)
  return jnp.concatenate([a, pad], axis=1).reshape(NSC * EPTP)


def _sc_degrees(ecatp, ones, zeros):
  @functools.partial(
      pl.kernel,
      out_type=jax.ShapeDtypeStruct((2 * NP, DH), jnp.float32),
      mesh=plsc.VectorSubcoreMesh(**_MESH),
      scratch_types=[
          pltpu.VMEM((CHUNK,), jnp.int32),
          pltpu.VMEM((CHUNK, DH), jnp.float32),
          pltpu.VMEM_SHARED((NP, DH), jnp.float32),
      ],
  )
  def k(ecat_h, ones_h, zeros_h, deg_h, idx_v, ones_v, acc):
    c = lax.axis_index("c")
    s = lax.axis_index("s")
    r0 = s * RPT
    pltpu.sync_copy(zeros_h.at[pl.ds(r0, RPT)], acc.at[pl.ds(r0, RPT)])
    pltpu.sync_copy(ones_h, ones_v)
    plsc.subcore_barrier()

    e0 = c * (NSC * EPTP) + s * EPTP

    def body(j, _):
      base = pl.multiple_of(e0 + j * CHUNK, CHUNK)
      pltpu.sync_copy(ecat_h.at[pl.ds(base, CHUNK)], idx_v)
      pltpu.sync_copy(ones_v, acc.at[idx_v], add=True)
      return ()

    lax.fori_loop(0, CPT, body, ())

    plsc.subcore_barrier()
    wb = pl.multiple_of(c * NP + r0, 8)
    pltpu.sync_copy(acc.at[pl.ds(r0, RPT)], deg_h.at[pl.ds(wb, RPT)])

  return k(ecatp, ones, zeros)


def _sc_aggregate(hs_cat, srcp, dstp, zeros):
  @functools.partial(
      pl.kernel,
      out_type=jax.ShapeDtypeStruct((2 * NP, DH), jnp.float32),
      mesh=plsc.VectorSubcoreMesh(**_MESH),
      scratch_types=[
          pltpu.VMEM((CHUNK,), jnp.int32),
          pltpu.VMEM((CHUNK,), jnp.int32),
          pltpu.VMEM((CHUNK, DH), jnp.float32),
          pltpu.VMEM_SHARED((NP, DH), jnp.float32),
          pltpu.SemaphoreType.DMA,
      ],
  )
  def k(hs_h, src_h, dst_h, zeros_h, out_h, sidx, didx, rows, acc, sem):
    c = lax.axis_index("c")
    s = lax.axis_index("s")
    cn = c * N
    r0 = s * RPT
    pltpu.sync_copy(zeros_h.at[pl.ds(r0, RPT)], acc.at[pl.ds(r0, RPT)])
    plsc.subcore_barrier()

    e0 = s * EPTP

    def body(j, _):
      base = pl.multiple_of(e0 + j * CHUNK, CHUNK)
      pltpu.sync_copy(src_h.at[pl.ds(base, CHUNK)], sidx)
      pltpu.sync_copy(dst_h.at[pl.ds(base, CHUNK)], didx)
      for q in range(CHUNK // 16):
        sl = pl.ds(q * 16, 16)
        sidx[sl] = sidx[sl] + cn
      pltpu.async_copy(hs_h.at[sidx], rows, sem).wait()
      pltpu.sync_copy(rows, acc.at[didx], add=True)
      return ()

    lax.fori_loop(0, CPT, body, ())

    plsc.subcore_barrier()
    wb = pl.multiple_of(c * NP + r0, 8)
    pltpu.sync_copy(acc.at[pl.ds(r0, RPT)], out_h.at[pl.ds(wb, RPT)])

  return k(hs_cat, srcp, dstp, zeros)


def _norm_from_deg(deg_col):
  return jnp.where(deg_col > 0.0,
                   lax.rsqrt(jnp.maximum(deg_col, 1e-12)), 0.0)


def _prep_body(x_ref, odeg_ref, hs_ref):
  norm = _norm_from_deg(odeg_ref[0, :, 0:1])
  hs = x_ref[...] * norm
  hs_ref[0] = hs[:, :DH]
  hs_ref[1] = hs[:, DH:]


def _tc_prep(x, deg3):
  return pl.pallas_call(
      _prep_body,
      grid=(GRID,),
      in_specs=[
          pl.BlockSpec((NB, D), lambda i: (i, 0)),
          pl.BlockSpec((1, NB, DH), lambda i: (0, i, 0)),
      ],
      out_specs=pl.BlockSpec((2, NB, DH), lambda i: (0, i, 0)),
      out_shape=jax.ShapeDtypeStruct((2, N, DH), jnp.float32),
  )(x, deg3)


def _conv_body(lo_ref, hi_ref, ideg_ref, w_ref, b_ref,
               y_ref, ssum_ref, ssq_ref):
  norm = _norm_from_deg(ideg_ref[0, :, 0:1])
  a = jnp.concatenate([lo_ref[0], hi_ref[0]], axis=1) * norm
  y = jnp.maximum(
      jnp.dot(a, w_ref[...], preferred_element_type=jnp.float32)
      + b_ref[...], 0.0)
  y_ref[...] = y

  @pl.when(pl.program_id(0) == 0)
  def _():
    ssum_ref[...] = jnp.zeros_like(ssum_ref)
    ssq_ref[...] = jnp.zeros_like(ssq_ref)

  ssum_ref[...] += jnp.sum(y, axis=0, keepdims=True)
  ssq_ref[...] += jnp.sum(y * y, axis=0, keepdims=True)


def _tc_conv(agg3, deg3, w, b2d):
  return pl.pallas_call(
      _conv_body,
      grid=(GRID,),
      in_specs=[
          pl.BlockSpec((1, NB, DH), lambda i: (0, i, 0)),
          pl.BlockSpec((1, NB, DH), lambda i: (1, i, 0)),
          pl.BlockSpec((1, NB, DH), lambda i: (1, i, 0)),
          pl.BlockSpec((D, D), lambda i: (0, 0)),
          pl.BlockSpec((1, D), lambda i: (0, 0)),
      ],
      out_specs=[
          pl.BlockSpec((NB, D), lambda i: (i, 0)),
          pl.BlockSpec((1, D), lambda i: (0, 0)),
          pl.BlockSpec((1, D), lambda i: (0, 0)),
      ],
      out_shape=[
          jax.ShapeDtypeStruct((N, D), jnp.float32),
          jax.ShapeDtypeStruct((1, D), jnp.float32),
          jax.ShapeDtypeStruct((1, D), jnp.float32),
      ],
  )(agg3, agg3, deg3, w, b2d)


def _bn(y, ssum_ref, ssq_ref, g_ref, be_ref):
  mu = ssum_ref[...] / N
  var = ssq_ref[...] / N - mu * mu
  return (y - mu) / jnp.sqrt(var + 1e-5) * g_ref[...] + be_ref[...]


def _bnscale_body(y_ref, ssum_ref, ssq_ref, g_ref, be_ref, odeg_ref,
                  hs_ref):
  h = _bn(y_ref[...], ssum_ref, ssq_ref, g_ref, be_ref)
  hs = h * _norm_from_deg(odeg_ref[0, :, 0:1])
  hs_ref[0] = hs[:, :DH]
  hs_ref[1] = hs[:, DH:]


def _tc_bnscale(y, ssum, ssq, g2d, be2d, deg3):
  return pl.pallas_call(
      _bnscale_body,
      grid=(GRID,),
      in_specs=[
          pl.BlockSpec((NB, D), lambda i: (i, 0)),
          pl.BlockSpec((1, D), lambda i: (0, 0)),
          pl.BlockSpec((1, D), lambda i: (0, 0)),
          pl.BlockSpec((1, D), lambda i: (0, 0)),
          pl.BlockSpec((1, D), lambda i: (0, 0)),
          pl.BlockSpec((1, NB, DH), lambda i: (0, i, 0)),
      ],
      out_specs=pl.BlockSpec((2, NB, DH), lambda i: (0, i, 0)),
      out_shape=jax.ShapeDtypeStruct((2, N, DH), jnp.float32),
  )(y, ssum, ssq, g2d, be2d, deg3)


def _readout_body(y_ref, ssum_ref, ssq_ref, g_ref, be_ref, wm_ref, n2g_ref,
                  bm_ref, out_ref):
  h = _bn(y_ref[...], ssum_ref, ssq_ref, g_ref, be_ref)
  t = jnp.dot(h, wm_ref[...], preferred_element_type=jnp.float32)[:, 0:1]
  gid = n2g_ref[:, 0:1]
  onehot = (gid == lax.broadcasted_iota(jnp.int32, (NB, G), 1)
            ).astype(jnp.float32)
  pg = jnp.sum(onehot * t, axis=0, keepdims=True)

  @pl.when(pl.program_id(0) == 0)
  def _():
    out_ref[...] = jnp.zeros_like(out_ref)

  out_ref[...] += pg

  @pl.when(pl.program_id(0) == GRID - 1)
  def _():
    out_ref[...] = jax.nn.sigmoid(out_ref[...] + bm_ref[...])


def _tc_readout(y2, ssum2, ssq2, g2d, be2d, wm_pad, n2g8, bm64):
  return pl.pallas_call(
      _readout_body,
      grid=(GRID,),
      in_specs=[
          pl.BlockSpec((NB, D), lambda i: (i, 0)),
          pl.BlockSpec((1, D), lambda i: (0, 0)),
          pl.BlockSpec((1, D), lambda i: (0, 0)),
          pl.BlockSpec((1, D), lambda i: (0, 0)),
          pl.BlockSpec((1, D), lambda i: (0, 0)),
          pl.BlockSpec((D, 128), lambda i: (0, 0)),
          pl.BlockSpec((NB, 8), lambda i: (i, 0)),
          pl.BlockSpec((1, G), lambda i: (0, 0)),
      ],
      out_specs=pl.BlockSpec((1, G), lambda i: (0, 0)),
      out_shape=jax.ShapeDtypeStruct((1, G), jnp.float32),
  )(y2, ssum2, ssq2, g2d, be2d, wm_pad, n2g8, bm64)


def kernel(x, W1, b1, g1, be1, W2, b2, g2, be2, Wm, bm, edge_index,
           node2graph):
  src = edge_index[0]
  dst = edge_index[1]
  srcp = _pad_idx(src, 0)
  dstp = _pad_idx(dst, NP - 1)
  ecatp = jnp.concatenate([_pad_idx(src, NP - 1), dstp])
  zeros = jnp.zeros((NP, DH), jnp.float32)
  ones = jnp.ones((CHUNK, DH), jnp.float32)

  deg3 = _sc_degrees(ecatp, ones, zeros).reshape(2, NP, DH)

  b1_2d = b1.reshape(1, D)
  g1_2d = g1.reshape(1, D)
  be1_2d = be1.reshape(1, D)
  b2_2d = b2.reshape(1, D)
  g2_2d = g2.reshape(1, D)
  be2_2d = be2.reshape(1, D)
  wm_pad = jnp.pad(Wm, ((0, 0), (0, 128 - Wm.shape[1])))
  n2g8 = jnp.broadcast_to(node2graph[:, None], (N, 8))
  bm64 = jnp.broadcast_to(bm.reshape(1, 1), (1, G))

  hs3 = _tc_prep(x, deg3)
  agg3 = _sc_aggregate(hs3.reshape(2 * N, DH), srcp, dstp, zeros
                       ).reshape(2, NP, DH)
  y1, ssum1, ssq1 = _tc_conv(agg3, deg3, W1, b1_2d)

  hs2_3 = _tc_bnscale(y1, ssum1, ssq1, g1_2d, be1_2d, deg3)
  agg2_3 = _sc_aggregate(hs2_3.reshape(2 * N, DH), srcp, dstp, zeros
                         ).reshape(2, NP, DH)
  y2, ssum2, ssq2 = _tc_conv(agg2_3, deg3, W2, b2_2d)

  out = _tc_readout(y2, ssum2, ssq2, g2_2d, be2_2d, wm_pad, n2g8, bm64)
  return out.reshape(G)


# speedup vs baseline: 4.0214x; 1.1394x over previous
import functools

import jax
import jax.numpy as jnp
from jax import lax
from jax.experimental import pallas as pl
from jax.experimental.pallas import tpu as pltpu
from jax.experimental.pallas import tpu_sc as plsc

N = 10000
NP = 10240
E = 160000
D = 256
DH = 128
G = 64
NB = 1000
GRID = N // NB

NSC = 16
EPT = E // NSC
CHUNK = 128
CPT = 80
EPTP = CPT * CHUNK
RPT = NP // NSC

_MESH = dict(core_axis_name="c", subcore_axis_name="s", num_cores=2,
             num_subcores=NSC)


def _pad_idx(a, fill):
  a = a.reshape(NSC, EPT)
  pad = jnp.full((NSC, EPTP - EPT), fill, jnp.int32)
  return jnp.concatenate([a, pad], axis=1).reshape(NSC * EPTP)


def _sc_degrees(ecc, ones, zeros):
  RING = 8

  @functools.partial(
      pl.kernel,
      out_type=jax.ShapeDtypeStruct((2 * NP, DH), jnp.float32),
      mesh=plsc.VectorSubcoreMesh(**_MESH),
      scratch_types=[
          pltpu.VMEM((CPT, CHUNK), jnp.int32),
          pltpu.VMEM((CHUNK, DH), jnp.float32),
          pltpu.VMEM_SHARED((NP, DH), jnp.float32),
          pltpu.SemaphoreType.DMA,
      ],
  )
  def k(ecc_h, ones_h, zeros_h, deg_h, idx_v, ones_v, acc, sem):
    c = lax.axis_index("c")
    s = lax.axis_index("s")
    r0 = s * RPT
    pltpu.sync_copy(zeros_h.at[pl.ds(r0, RPT)], acc.at[pl.ds(r0, RPT)])
    pltpu.sync_copy(ones_h, ones_v)
    row0 = pl.multiple_of((c * NSC + s) * CPT, 8)
    pltpu.sync_copy(ecc_h.at[pl.ds(row0, CPT)], idx_v)
    plsc.subcore_barrier()

    def fire(j):
      pltpu.async_copy(ones_v, acc.at[idx_v.at[j]], sem, add=True)

    def drain():
      pltpu.make_async_copy(zeros_h.at[pl.ds(0, CHUNK)], ones_v, sem).wait()

    for q in range(RING):
      fire(q)

    def body(i, _):
      drain()
      fire(i + RING)
      return ()

    lax.fori_loop(0, CPT - RING, body, ())
    for _ in range(RING):
      drain()

    plsc.subcore_barrier()
    wb = pl.multiple_of(c * NP + r0, 8)
    pltpu.sync_copy(acc.at[pl.ds(r0, RPT)], deg_h.at[pl.ds(wb, RPT)])

  return k(ecc, ones, zeros)


def _sc_aggregate(hs_cat, scc, dcc, zeros):
  HCPT = CPT // 2

  @functools.partial(
      pl.kernel,
      out_type=jax.ShapeDtypeStruct((2 * NP, DH), jnp.float32),
      mesh=plsc.VectorSubcoreMesh(**_MESH),
      scratch_types=[
          pltpu.VMEM((CPT, CHUNK), jnp.int32),
          pltpu.VMEM((HCPT, CHUNK), jnp.int32),
          pltpu.VMEM((CHUNK, DH), jnp.float32),
          pltpu.VMEM((CHUNK, DH), jnp.float32),
          pltpu.VMEM_SHARED((NP, DH), jnp.float32),
          pltpu.SemaphoreType.DMA,
          pltpu.SemaphoreType.DMA,
      ],
  )
  def k(hs_h, scc_h, dcc_h, zeros_h, out_h,
        sidx_v, didx_v, rows_a, rows_b, acc, sem_a, sem_b):
    c = lax.axis_index("c")
    s = lax.axis_index("s")
    r0 = s * RPT
    pltpu.sync_copy(zeros_h.at[pl.ds(r0, RPT)], acc.at[pl.ds(r0, RPT)])
    srow0 = pl.multiple_of((c * NSC + s) * CPT, 8)
    pltpu.sync_copy(scc_h.at[pl.ds(srow0, CPT)], sidx_v)
    plsc.subcore_barrier()

    def gstart(j, buf, sem):
      pltpu.async_copy(hs_h.at[sidx_v.at[j]], buf, sem)

    def gwait(buf, sem):
      pltpu.make_async_copy(hs_h.at[pl.ds(0, CHUNK)], buf, sem).wait()

    def sadd(jloc, buf):
      pltpu.sync_copy(buf, acc.at[didx_v.at[jloc]], add=True)

    gstart(0, rows_a, sem_a)

    for p in (0, 1):
      dr = pl.multiple_of(s * CPT + p * HCPT, 8)
      pltpu.sync_copy(dcc_h.at[pl.ds(dr, HCPT)], didx_v)

      def body(i, _, p=p):
        ja = p * HCPT + 2 * i
        gwait(rows_a, sem_a)
        gstart(ja + 1, rows_b, sem_b)
        sadd(2 * i, rows_a)
        gwait(rows_b, sem_b)
        gstart(ja + 2, rows_a, sem_a)
        sadd(2 * i + 1, rows_b)
        return ()

      lax.fori_loop(0, HCPT // 2 if p == 0 else HCPT // 2 - 1, body, ())

    gwait(rows_a, sem_a)
    gstart(CPT - 1, rows_b, sem_b)
    sadd(HCPT - 2, rows_a)
    gwait(rows_b, sem_b)
    sadd(HCPT - 1, rows_b)

    plsc.subcore_barrier()
    wb = pl.multiple_of(c * NP + r0, 8)
    pltpu.sync_copy(acc.at[pl.ds(r0, RPT)], out_h.at[pl.ds(wb, RPT)])

  return k(hs_cat, scc, dcc, zeros)


def _norm_from_deg(deg_col):
  return jnp.where(deg_col > 0.0,
                   lax.rsqrt(jnp.maximum(deg_col, 1e-12)), 0.0)


def _prep_body(x_ref, odeg_ref, hs_ref):
  norm = _norm_from_deg(odeg_ref[0, :, 0:1])
  hs = x_ref[...] * norm
  hs_ref[0] = hs[:, :DH]
  hs_ref[1] = hs[:, DH:]


def _tc_prep(x, deg3):
  return pl.pallas_call(
      _prep_body,
      grid=(GRID,),
      in_specs=[
          pl.BlockSpec((NB, D), lambda i: (i, 0)),
          pl.BlockSpec((1, NB, DH), lambda i: (0, i, 0)),
      ],
      out_specs=pl.BlockSpec((2, NB, DH), lambda i: (0, i, 0)),
      out_shape=jax.ShapeDtypeStruct((2, N, DH), jnp.float32),
  )(x, deg3)


def _conv_body(lo_ref, hi_ref, ideg_ref, w_ref, b_ref,
               y_ref, ssum_ref, ssq_ref):
  norm = _norm_from_deg(ideg_ref[0, :, 0:1])
  a = jnp.concatenate([lo_ref[0], hi_ref[0]], axis=1) * norm
  y = jnp.maximum(
      jnp.dot(a, w_ref[...], preferred_element_type=jnp.float32)
      + b_ref[...], 0.0)
  y_ref[...] = y

  @pl.when(pl.program_id(0) == 0)
  def _():
    ssum_ref[...] = jnp.zeros_like(ssum_ref)
    ssq_ref[...] = jnp.zeros_like(ssq_ref)

  ssum_ref[...] += jnp.sum(y, axis=0, keepdims=True)
  ssq_ref[...] += jnp.sum(y * y, axis=0, keepdims=True)


def _tc_conv(agg3, deg3, w, b2d):
  return pl.pallas_call(
      _conv_body,
      grid=(GRID,),
      in_specs=[
          pl.BlockSpec((1, NB, DH), lambda i: (0, i, 0)),
          pl.BlockSpec((1, NB, DH), lambda i: (1, i, 0)),
          pl.BlockSpec((1, NB, DH), lambda i: (1, i, 0)),
          pl.BlockSpec((D, D), lambda i: (0, 0)),
          pl.BlockSpec((1, D), lambda i: (0, 0)),
      ],
      out_specs=[
          pl.BlockSpec((NB, D), lambda i: (i, 0)),
          pl.BlockSpec((1, D), lambda i: (0, 0)),
          pl.BlockSpec((1, D), lambda i: (0, 0)),
      ],
      out_shape=[
          jax.ShapeDtypeStruct((N, D), jnp.float32),
          jax.ShapeDtypeStruct((1, D), jnp.float32),
          jax.ShapeDtypeStruct((1, D), jnp.float32),
      ],
  )(agg3, agg3, deg3, w, b2d)


def _bn(y, ssum_ref, ssq_ref, g_ref, be_ref):
  mu = ssum_ref[...] / N
  var = ssq_ref[...] / N - mu * mu
  return (y - mu) / jnp.sqrt(var + 1e-5) * g_ref[...] + be_ref[...]


def _bnscale_body(y_ref, ssum_ref, ssq_ref, g_ref, be_ref, odeg_ref,
                  hs_ref):
  h = _bn(y_ref[...], ssum_ref, ssq_ref, g_ref, be_ref)
  hs = h * _norm_from_deg(odeg_ref[0, :, 0:1])
  hs_ref[0] = hs[:, :DH]
  hs_ref[1] = hs[:, DH:]


def _tc_bnscale(y, ssum, ssq, g2d, be2d, deg3):
  return pl.pallas_call(
      _bnscale_body,
      grid=(GRID,),
      in_specs=[
          pl.BlockSpec((NB, D), lambda i: (i, 0)),
          pl.BlockSpec((1, D), lambda i: (0, 0)),
          pl.BlockSpec((1, D), lambda i: (0, 0)),
          pl.BlockSpec((1, D), lambda i: (0, 0)),
          pl.BlockSpec((1, D), lambda i: (0, 0)),
          pl.BlockSpec((1, NB, DH), lambda i: (0, i, 0)),
      ],
      out_specs=pl.BlockSpec((2, NB, DH), lambda i: (0, i, 0)),
      out_shape=jax.ShapeDtypeStruct((2, N, DH), jnp.float32),
  )(y, ssum, ssq, g2d, be2d, deg3)


def _readout_body(y_ref, ssum_ref, ssq_ref, g_ref, be_ref, wm_ref, n2g_ref,
                  bm_ref, out_ref):
  h = _bn(y_ref[...], ssum_ref, ssq_ref, g_ref, be_ref)
  t = jnp.dot(h, wm_ref[...], preferred_element_type=jnp.float32)[:, 0:1]
  gid = n2g_ref[:, 0:1]
  onehot = (gid == lax.broadcasted_iota(jnp.int32, (NB, G), 1)
            ).astype(jnp.float32)
  pg = jnp.sum(onehot * t, axis=0, keepdims=True)

  @pl.when(pl.program_id(0) == 0)
  def _():
    out_ref[...] = jnp.zeros_like(out_ref)

  out_ref[...] += pg

  @pl.when(pl.program_id(0) == GRID - 1)
  def _():
    out_ref[...] = jax.nn.sigmoid(out_ref[...] + bm_ref[...])


def _tc_readout(y2, ssum2, ssq2, g2d, be2d, wm_pad, n2g8, bm64):
  return pl.pallas_call(
      _readout_body,
      grid=(GRID,),
      in_specs=[
          pl.BlockSpec((NB, D), lambda i: (i, 0)),
          pl.BlockSpec((1, D), lambda i: (0, 0)),
          pl.BlockSpec((1, D), lambda i: (0, 0)),
          pl.BlockSpec((1, D), lambda i: (0, 0)),
          pl.BlockSpec((1, D), lambda i: (0, 0)),
          pl.BlockSpec((D, 128), lambda i: (0, 0)),
          pl.BlockSpec((NB, 8), lambda i: (i, 0)),
          pl.BlockSpec((1, G), lambda i: (0, 0)),
      ],
      out_specs=pl.BlockSpec((1, G), lambda i: (0, 0)),
      out_shape=jax.ShapeDtypeStruct((1, G), jnp.float32),
  )(y2, ssum2, ssq2, g2d, be2d, wm_pad, n2g8, bm64)


def kernel(x, W1, b1, g1, be1, W2, b2, g2, be2, Wm, bm, edge_index,
           node2graph):
  src = edge_index[0]
  dst = edge_index[1]
  srcp = _pad_idx(src, 0)
  dstp = _pad_idx(dst, NP - 1)
  scc = jnp.concatenate([srcp, srcp + N]).reshape(2 * NSC * CPT, CHUNK)
  dcc = dstp.reshape(NSC * CPT, CHUNK)
  ecc = jnp.concatenate([_pad_idx(src, NP - 1), dstp]
                        ).reshape(2 * NSC * CPT, CHUNK)
  zeros = jnp.zeros((NP, DH), jnp.float32)
  ones = jnp.ones((CHUNK, DH), jnp.float32)

  deg3 = _sc_degrees(ecc, ones, zeros).reshape(2, NP, DH)

  b1_2d = b1.reshape(1, D)
  g1_2d = g1.reshape(1, D)
  be1_2d = be1.reshape(1, D)
  b2_2d = b2.reshape(1, D)
  g2_2d = g2.reshape(1, D)
  be2_2d = be2.reshape(1, D)
  wm_pad = jnp.pad(Wm, ((0, 0), (0, 128 - Wm.shape[1])))
  n2g8 = jnp.broadcast_to(node2graph[:, None], (N, 8))
  bm64 = jnp.broadcast_to(bm.reshape(1, 1), (1, G))

  hs3 = _tc_prep(x, deg3)
  agg3 = _sc_aggregate(hs3.reshape(2 * N, DH), scc, dcc, zeros
                       ).reshape(2, NP, DH)
  y1, ssum1, ssq1 = _tc_conv(agg3, deg3, W1, b1_2d)

  hs2_3 = _tc_bnscale(y1, ssum1, ssq1, g1_2d, be1_2d, deg3)
  agg2_3 = _sc_aggregate(hs2_3.reshape(2 * N, DH), scc, dcc, zeros
                         ).reshape(2, NP, DH)
  y2, ssum2, ssq2 = _tc_conv(agg2_3, deg3, W2, b2_2d)

  out = _tc_readout(y2, ssum2, ssq2, g2_2d, be2_2d, wm_pad, n2g8, bm64)
  return out.reshape(G)
